# Initial kernel scaffold; baseline (speedup 1.0000x reference)
import jax
import jax.numpy as jnp
from jax.experimental import pallas as pl


def kernel(queries, heads, facts, entity_degrees, query_emb, entity_emb, q_Wih, q_Whh, q_bih, q_bhh, e_Wih, e_Whh, e_bih, e_bhh, qlin_W, qlin_b, elin_W, elin_b):
    raise NotImplementedError("write your pallas kernel here")


# speedup vs baseline: 53.8970x; 53.8970x over previous
import functools

import jax
import jax.numpy as jnp
from jax import lax
from jax.experimental import pallas as pl
from jax.experimental.pallas import tpu as pltpu
from jax.experimental.pallas import tpu_sc as plsc

N_ENT = 10000
N_OPS = 24
RANK = 3
STEPS = 3
B = 64
NF = 50000
DLEN = 8
EMB = 128
HID = 128

NTILE = 32
TPF = 1568
NF_PAD = NTILE * TPF
CH = 2000
NCH = NF // CH
EBLK = 1000

_SC_MESH = plsc.VectorSubcoreMesh(core_axis_name="c", subcore_axis_name="s")
_SC_PARAMS = pltpu.CompilerParams(needs_layout_passes=False)


def _entity_body(deg_ref, emb_ref, wih_ref, whh_ref, b_ref, elin_ref, elb_ref,
                 out_ref):
    f32 = jnp.float32
    deg = deg_ref[...]
    emb = emb_ref[...]

    xs = []
    for t in range(DLEN):
        oh = (deg[:, t:t + 1]
              == lax.broadcasted_iota(jnp.int32, (EBLK, 32), 1)).astype(f32)
        xs.append(jnp.dot(oh, emb, preferred_element_type=f32))

    hs = []
    for d in range(2):
        wih = wih_ref[d]
        whh = whh_ref[d]
        bias = b_ref[d, 0:1, :]
        h = jnp.zeros((EBLK, HID), f32)
        c = jnp.zeros((EBLK, HID), f32)
        for s in range(DLEN):
            x = xs[s] if d == 0 else xs[DLEN - 1 - s]
            g = (jnp.dot(x, wih, preferred_element_type=f32)
                 + jnp.dot(h, whh, preferred_element_type=f32) + bias)
            gi = jax.nn.sigmoid(g[:, 0:HID])
            gf = jax.nn.sigmoid(g[:, HID:2 * HID])
            gg = jnp.tanh(g[:, 2 * HID:3 * HID])
            go = jax.nn.sigmoid(g[:, 3 * HID:4 * HID])
            c = gf * c + gi * gg
            h = go * jnp.tanh(c)
        hs.append(h)
    hT = jnp.concatenate(hs, axis=1)
    logits = (jnp.dot(hT, elin_ref[...], preferred_element_type=f32)
              + elb_ref[0:1, :])
    m = jnp.max(logits, axis=1, keepdims=True)
    e = jnp.exp(logits - m)
    out_ref[...] = e / jnp.sum(e, axis=1, keepdims=True)


def _entity_lstm_call(entity_degrees, emb_pad, wihT, whhT, ebias, elinT, elbP):
    return pl.pallas_call(
        _entity_body,
        grid=(N_ENT // EBLK,),
        in_specs=[
            pl.BlockSpec((EBLK, DLEN), lambda i: (i, 0)),
            pl.BlockSpec((32, EMB), lambda i: (0, 0)),
            pl.BlockSpec((2, EMB, 4 * HID), lambda i: (0, 0, 0)),
            pl.BlockSpec((2, HID, 4 * HID), lambda i: (0, 0, 0)),
            pl.BlockSpec((2, 8, 4 * HID), lambda i: (0, 0, 0)),
            pl.BlockSpec((2 * HID, N_OPS), lambda i: (0, 0)),
            pl.BlockSpec((8, N_OPS), lambda i: (0, 0)),
        ],
        out_specs=pl.BlockSpec((EBLK, N_OPS), lambda i: (i, 0)),
        out_shape=jax.ShapeDtypeStruct((N_ENT, N_OPS), jnp.float32),
    )(entity_degrees, emb_pad, wihT, whhT, ebias, elinT, elbP)


def _query_body(ohT_ref, embT_ref, wih_ref, whh_ref, b_ref, qlin_ref, qlb_ref,
                out_ref):
    f32 = jnp.float32
    xT = jnp.dot(embT_ref[...], ohT_ref[...], preferred_element_type=f32)
    for r in range(RANK):
        hs = []
        for d in range(2):
            wih = wih_ref[r, d]
            whh = whh_ref[r, d]
            bias = b_ref[r, d, :, 0:1]
            h = jnp.zeros((HID, B), f32)
            c = jnp.zeros((HID, B), f32)
            hd = []
            for _ in range(STEPS):
                g = (jnp.dot(wih, xT, preferred_element_type=f32)
                     + jnp.dot(whh, h, preferred_element_type=f32) + bias)
                gi = jax.nn.sigmoid(g[0:HID])
                gf = jax.nn.sigmoid(g[HID:2 * HID])
                gg = jnp.tanh(g[2 * HID:3 * HID])
                go = jax.nn.sigmoid(g[3 * HID:4 * HID])
                c = gf * c + gi * gg
                h = go * jnp.tanh(c)
                hd.append(h)
            hs.append(hd)
        for t in range(STEPS):
            outT = jnp.concatenate([hs[0][t], hs[1][STEPS - 1 - t]], axis=0)
            lg = (jnp.dot(qlin_ref[...], outT, preferred_element_type=f32)
                  + qlb_ref[:, 0:1])
            m = jnp.max(lg, axis=0, keepdims=True)
            e = jnp.exp(lg - m)
            out_ref[r * STEPS + t] = e / jnp.sum(e, axis=0, keepdims=True)


def _query_attn_call(q_ohT, q_embT, q_Wih, q_Whh, qb, qlinP, qlbP):
    return pl.pallas_call(
        _query_body,
        in_specs=[
            pl.BlockSpec((32, B), lambda: (0, 0)),
            pl.BlockSpec((EMB, 32), lambda: (0, 0)),
            pl.BlockSpec((RANK, 2, 4 * HID, EMB), lambda: (0, 0, 0, 0)),
            pl.BlockSpec((RANK, 2, 4 * HID, HID), lambda: (0, 0, 0, 0)),
            pl.BlockSpec((RANK, 2, 4 * HID, 8), lambda: (0, 0, 0, 0)),
            pl.BlockSpec((32, 2 * HID), lambda: (0, 0)),
            pl.BlockSpec((32, 8), lambda: (0, 0)),
        ],
        out_specs=pl.BlockSpec((RANK * STEPS, 32, B), lambda: (0, 0, 0)),
        out_shape=jax.ShapeDtypeStruct((RANK * STEPS, 32, B), jnp.float32),
    )(q_ohT, q_embT, q_Wih, q_Whh, qb, qlinP, qlbP)


@functools.partial(
    pl.kernel,
    out_type=(jax.ShapeDtypeStruct((NF_PAD,), jnp.int32),
              jax.ShapeDtypeStruct((NF_PAD,), jnp.float32)),
    mesh=_SC_MESH,
    compiler_params=_SC_PARAMS,
    scratch_types=[
        pltpu.VMEM((TPF * 3,), jnp.int32),
        pltpu.VMEM((TPF,), jnp.int32),
        pltpu.VMEM((TPF,), jnp.int32),
        pltpu.VMEM((TPF,), jnp.float32),
        pltpu.SemaphoreType.DMA,
    ],
)
def _pack_kernel(facts_hbm, ea_hbm, pk_hbm, vl_hbm, f_v, idx_v, pk_v, vl_v,
                 sem):
    wid = lax.axis_index("s") * 2 + lax.axis_index("c")
    base = wid * TPF
    pltpu.sync_copy(facts_hbm.at[pl.ds(base * 3, TPF * 3)], f_v)
    lanes = lax.iota(jnp.int32, 16)

    def grp(g, carry):
        row = (g * 16 + lanes) * 3
        rel = plsc.load_gather(f_v, [row])
        fh = plsc.load_gather(f_v, [row + 1])
        ft = plsc.load_gather(f_v, [row + 2])
        relc = jnp.minimum(rel, 12)
        pk_v[pl.ds(g * 16, 16)] = fh | (ft << 14) | (relc << 28)
        idx_v[pl.ds(g * 16, 16)] = fh * N_OPS + rel
        return carry

    lax.fori_loop(0, TPF // 16, grp, 0)
    pltpu.async_copy(ea_hbm.at[idx_v], vl_v, sem).wait()
    pltpu.sync_copy(pk_v, pk_hbm.at[pl.ds(base, TPF)])
    pltpu.sync_copy(vl_v, vl_hbm.at[pl.ds(base, TPF)])


@functools.partial(
    pl.kernel,
    out_type=jax.ShapeDtypeStruct((B, N_ENT), jnp.float32),
    mesh=_SC_MESH,
    compiler_params=_SC_PARAMS,
    scratch_types=[
        pltpu.VMEM((N_ENT,), jnp.float32),
        pltpu.VMEM((N_ENT,), jnp.float32),
        pltpu.VMEM((N_ENT,), jnp.float32),
        pltpu.VMEM((N_ENT,), jnp.float32),
        pltpu.VMEM((N_ENT,), jnp.float32),
        pltpu.VMEM((N_ENT,), jnp.float32),
        pltpu.VMEM((2 * CH,), jnp.int32),
        pltpu.VMEM((2 * CH,), jnp.float32),
        pltpu.VMEM((64,), jnp.float32),
        pltpu.VMEM((B,), jnp.int32),
        pltpu.SemaphoreType.DMA,
        pltpu.SemaphoreType.DMA,
    ],
)
def _prop_kernel(pk_hbm, vl_hbm, tbl_hbm, heads_hbm, out_hbm,
                 M0, M1, A0, A1, L0, L1, pk_v, vl_v, tbl_v, heads_v,
                 sem_pk, sem_vl):
    f32 = jnp.float32
    wid = lax.axis_index("s") * 2 + lax.axis_index("c")
    lanes = lax.iota(jnp.int32, 16)
    lane0 = lanes == 0
    NV = N_ENT // 16
    z16 = jnp.zeros(16, f32)
    ones16 = jnp.ones(16, f32)

    pltpu.sync_copy(heads_hbm, heads_v)
    h0v = plsc.load_gather(heads_v, [jnp.full((16,), 0, jnp.int32) + 2 * wid])
    h1v = plsc.load_gather(heads_v,
                           [jnp.full((16,), 1, jnp.int32) + 2 * wid])

    def zero_pass(i, carry):
        L0[pl.ds(i * 16, 16)] = z16
        L1[pl.ds(i * 16, 16)] = z16
        return carry

    lax.fori_loop(0, NV, zero_pass, 0)

    for r in range(RANK):
        def minit(i, carry):
            M0[pl.ds(i * 16, 16)] = z16
            M1[pl.ds(i * 16, 16)] = z16
            return carry

        lax.fori_loop(0, NV, minit, 0)
        plsc.store_scatter(M0, [h0v], ones16, mask=lane0)
        plsc.store_scatter(M1, [h1v], ones16, mask=lane0)
        sm0 = ones16
        sm1 = ones16

        for t in range(STEPS):
            ti = r * STEPS + t
            pltpu.sync_copy(tbl_hbm.at[ti * NTILE + wid], tbl_v)
            a24_0 = plsc.load_gather(tbl_v, [jnp.full((16,), 15, jnp.int32)])
            a24_1 = plsc.load_gather(tbl_v, [jnp.full((16,), 47, jnp.int32)])

            def ainit(i, carry):
                A0[pl.ds(i * 16, 16)] = a24_0 * M0[pl.ds(i * 16, 16)]
                A1[pl.ds(i * 16, 16)] = a24_1 * M1[pl.ds(i * 16, 16)]
                return carry

            lax.fori_loop(0, NV, ainit, 0)

            pltpu.make_async_copy(pk_hbm.at[pl.ds(0, CH)],
                                  pk_v.at[pl.ds(0, CH)], sem_pk).start()
            pltpu.make_async_copy(vl_hbm.at[pl.ds(0, CH)],
                                  vl_v.at[pl.ds(0, CH)], sem_vl).start()

            def chunk(ci, accs):
                acc0, acc1 = accs
                par = lax.rem(ci, 2) * CH
                pltpu.make_async_copy(pk_hbm.at[pl.ds(0, CH)],
                                      pk_v.at[pl.ds(par, CH)], sem_pk).wait()
                pltpu.make_async_copy(vl_hbm.at[pl.ds(0, CH)],
                                      vl_v.at[pl.ds(par, CH)], sem_vl).wait()

                @pl.when(ci + 1 < NCH)
                def _():
                    pltpu.make_async_copy(
                        pk_hbm.at[pl.ds((ci + 1) * CH, CH)],
                        pk_v.at[pl.ds(CH - par, CH)], sem_pk).start()
                    pltpu.make_async_copy(
                        vl_hbm.at[pl.ds((ci + 1) * CH, CH)],
                        vl_v.at[pl.ds(CH - par, CH)], sem_vl).start()

                def grp(g, a):
                    a0, a1 = a
                    pk = pk_v[pl.ds(par + g * 16, 16)]
                    vl = vl_v[pl.ds(par + g * 16, 16)]
                    fh = pk & 0x3FFF
                    ft = (pk >> 14) & 0x3FFF
                    rl = (pk >> 28) & 0xF
                    cf0 = vl * plsc.load_gather(tbl_v, [rl])
                    cb0 = vl * plsc.load_gather(tbl_v, [rl + 16])
                    cf1 = vl * plsc.load_gather(tbl_v, [rl + 32])
                    cb1 = vl * plsc.load_gather(tbl_v, [rl + 48])
                    m0h = plsc.load_gather(M0, [fh])
                    m0t = plsc.load_gather(M0, [ft])
                    m1h = plsc.load_gather(M1, [fh])
                    m1t = plsc.load_gather(M1, [ft])
                    p0f = m0h * cf0
                    p0b = m0t * cb0
                    p1f = m1h * cf1
                    p1b = m1t * cb1
                    plsc.addupdate_scatter(A0, [ft], p0f)
                    plsc.addupdate_scatter(A0, [fh], p0b)
                    plsc.addupdate_scatter(A1, [ft], p1f)
                    plsc.addupdate_scatter(A1, [fh], p1b)
                    return (a0 + p0f + p0b, a1 + p1f + p1b)

                return lax.fori_loop(0, CH // 16, grp, (acc0, acc1))

            acc0, acc1 = lax.fori_loop(0, NCH, chunk, (z16, z16))

            nr0 = a24_0 * sm0 + jnp.broadcast_to(jnp.sum(acc0), (16,))
            nr1 = a24_1 * sm1 + jnp.broadcast_to(jnp.sum(acc1), (16,))
            rc0 = 1.0 / jnp.maximum(nr0, 1e-20)
            rc1 = 1.0 / jnp.maximum(nr1, 1e-20)
            sm0 = nr0 * rc0
            sm1 = nr1 * rc1

            if t == STEPS - 1:
                def renorm_acc(i, carry):
                    m0 = A0[pl.ds(i * 16, 16)] * rc0
                    m1 = A1[pl.ds(i * 16, 16)] * rc1
                    M0[pl.ds(i * 16, 16)] = m0
                    M1[pl.ds(i * 16, 16)] = m1
                    L0[pl.ds(i * 16, 16)] = L0[pl.ds(i * 16, 16)] + m0
                    L1[pl.ds(i * 16, 16)] = L1[pl.ds(i * 16, 16)] + m1
                    return carry

                lax.fori_loop(0, NV, renorm_acc, 0)
            else:
                def renorm(i, carry):
                    M0[pl.ds(i * 16, 16)] = A0[pl.ds(i * 16, 16)] * rc0
                    M1[pl.ds(i * 16, 16)] = A1[pl.ds(i * 16, 16)] * rc1
                    return carry

                lax.fori_loop(0, NV, renorm, 0)

    pltpu.sync_copy(L0, out_hbm.at[2 * wid])
    pltpu.sync_copy(L1, out_hbm.at[2 * wid + 1])


def kernel(queries, heads, facts, entity_degrees, query_emb, entity_emb,
           q_Wih, q_Whh, q_bih, q_bhh, e_Wih, e_Whh, e_bih, e_bhh,
           qlin_W, qlin_b, elin_W, elin_b):
    f32 = jnp.float32

    emb_pad = jnp.zeros((32, EMB), f32).at[:N_OPS + 1].set(entity_emb)
    wihT = jnp.transpose(e_Wih, (0, 2, 1))
    whhT = jnp.transpose(e_Whh, (0, 2, 1))
    ebias = jnp.zeros((2, 8, 4 * HID), f32).at[:, 0, :].set(e_bih + e_bhh)
    elinT = jnp.transpose(elin_W)
    elbP = jnp.zeros((8, N_OPS), f32).at[0].set(elin_b)
    ea = _entity_lstm_call(entity_degrees.astype(jnp.int32), emb_pad, wihT,
                           whhT, ebias, elinT, elbP)

    q_ohT = jax.nn.one_hot(queries, 32, axis=0, dtype=f32)
    q_embT = jnp.zeros((EMB, 32), f32).at[:, :N_OPS].set(query_emb.T)
    qb = jnp.zeros((RANK, 2, 4 * HID, 8), f32).at[:, :, :, 0].set(
        q_bih + q_bhh)
    qlinP = jnp.zeros((32, 2 * HID), f32).at[:N_OPS + 1].set(qlin_W)
    qlbP = jnp.full((32, 8), -1e30, f32).at[:N_OPS + 1, :].set(
        qlin_b[:, None])
    attnT = _query_attn_call(q_ohT, q_embT, q_Wih, q_Whh, qb, qlinP, qlbP)

    fwd = attnT[:, 0:12, :]
    bwd = attnT[:, 12:24, :]
    a24 = attnT[:, 24:25, :]
    zero3 = jnp.zeros((RANK * STEPS, 3, B), f32)
    zero4 = jnp.zeros((RANK * STEPS, 4, B), f32)
    tbl = jnp.concatenate([fwd, zero3, a24, bwd, zero4], axis=1)
    tbl = jnp.transpose(tbl, (0, 2, 1)).reshape(RANK * STEPS * NTILE, 64)

    facts_pad = jnp.zeros((NF_PAD, 3), jnp.int32).at[:NF].set(
        facts.astype(jnp.int32))
    pk, vl = _pack_kernel(facts_pad.reshape(-1), ea.reshape(-1))

    return _prop_kernel(pk, vl, tbl, heads.astype(jnp.int32))


# speedup vs baseline: 73.7164x; 1.3677x over previous
import functools

import jax
import jax.numpy as jnp
from jax import lax
from jax.experimental import pallas as pl
from jax.experimental.pallas import tpu as pltpu
from jax.experimental.pallas import tpu_sc as plsc

N_ENT = 10000
N_OPS = 24
RANK = 3
STEPS = 3
B = 64
NF = 50000
DLEN = 8
EMB = 128
HID = 128

NTILE = 32
TPF = 1568
NF_PAD = NTILE * TPF
CH = 10000
NCH = NF // CH
EBLK = 1000

_SC_MESH = plsc.VectorSubcoreMesh(core_axis_name="c", subcore_axis_name="s")
_SC_PARAMS = pltpu.CompilerParams(needs_layout_passes=False)


def _entity_body(deg_ref, emb_ref, wih_ref, whh_ref, b_ref, elin_ref, elb_ref,
                 out_ref):
    f32 = jnp.float32
    deg = deg_ref[...]
    emb = emb_ref[...]

    xs = []
    for t in range(DLEN):
        oh = (deg[:, t:t + 1]
              == lax.broadcasted_iota(jnp.int32, (EBLK, 32), 1)).astype(f32)
        xs.append(jnp.dot(oh, emb, preferred_element_type=f32))

    hs = []
    for d in range(2):
        wih = wih_ref[d]
        whh = whh_ref[d]
        bias = b_ref[d, 0:1, :]
        h = jnp.zeros((EBLK, HID), f32)
        c = jnp.zeros((EBLK, HID), f32)
        for s in range(DLEN):
            x = xs[s] if d == 0 else xs[DLEN - 1 - s]
            g = (jnp.dot(x, wih, preferred_element_type=f32)
                 + jnp.dot(h, whh, preferred_element_type=f32) + bias)
            gi = jax.nn.sigmoid(g[:, 0:HID])
            gf = jax.nn.sigmoid(g[:, HID:2 * HID])
            gg = jnp.tanh(g[:, 2 * HID:3 * HID])
            go = jax.nn.sigmoid(g[:, 3 * HID:4 * HID])
            c = gf * c + gi * gg
            h = go * jnp.tanh(c)
        hs.append(h)
    hT = jnp.concatenate(hs, axis=1)
    logits = (jnp.dot(hT, elin_ref[...], preferred_element_type=f32)
              + elb_ref[0:1, :])
    m = jnp.max(logits, axis=1, keepdims=True)
    e = jnp.exp(logits - m)
    out_ref[...] = e / jnp.sum(e, axis=1, keepdims=True)


def _entity_lstm_call(entity_degrees, emb_pad, wihT, whhT, ebias, elinT, elbP):
    return pl.pallas_call(
        _entity_body,
        grid=(N_ENT // EBLK,),
        in_specs=[
            pl.BlockSpec((EBLK, DLEN), lambda i: (i, 0)),
            pl.BlockSpec((32, EMB), lambda i: (0, 0)),
            pl.BlockSpec((2, EMB, 4 * HID), lambda i: (0, 0, 0)),
            pl.BlockSpec((2, HID, 4 * HID), lambda i: (0, 0, 0)),
            pl.BlockSpec((2, 8, 4 * HID), lambda i: (0, 0, 0)),
            pl.BlockSpec((2 * HID, N_OPS), lambda i: (0, 0)),
            pl.BlockSpec((8, N_OPS), lambda i: (0, 0)),
        ],
        out_specs=pl.BlockSpec((EBLK, N_OPS), lambda i: (i, 0)),
        out_shape=jax.ShapeDtypeStruct((N_ENT, N_OPS), jnp.float32),
    )(entity_degrees, emb_pad, wihT, whhT, ebias, elinT, elbP)


def _query_body(ohT_ref, embT_ref, wih_ref, whh_ref, b_ref, qlin_ref, qlb_ref,
                out_ref):
    f32 = jnp.float32
    xT = jnp.dot(embT_ref[...], ohT_ref[...], preferred_element_type=f32)
    for r in range(RANK):
        hs = []
        for d in range(2):
            wih = wih_ref[r, d]
            whh = whh_ref[r, d]
            bias = b_ref[r, d, :, 0:1]
            h = jnp.zeros((HID, B), f32)
            c = jnp.zeros((HID, B), f32)
            hd = []
            for _ in range(STEPS):
                g = (jnp.dot(wih, xT, preferred_element_type=f32)
                     + jnp.dot(whh, h, preferred_element_type=f32) + bias)
                gi = jax.nn.sigmoid(g[0:HID])
                gf = jax.nn.sigmoid(g[HID:2 * HID])
                gg = jnp.tanh(g[2 * HID:3 * HID])
                go = jax.nn.sigmoid(g[3 * HID:4 * HID])
                c = gf * c + gi * gg
                h = go * jnp.tanh(c)
                hd.append(h)
            hs.append(hd)
        for t in range(STEPS):
            outT = jnp.concatenate([hs[0][t], hs[1][STEPS - 1 - t]], axis=0)
            lg = (jnp.dot(qlin_ref[...], outT, preferred_element_type=f32)
                  + qlb_ref[:, 0:1])
            m = jnp.max(lg, axis=0, keepdims=True)
            e = jnp.exp(lg - m)
            out_ref[r * STEPS + t] = e / jnp.sum(e, axis=0, keepdims=True)


def _query_attn_call(q_ohT, q_embT, q_Wih, q_Whh, qb, qlinP, qlbP):
    return pl.pallas_call(
        _query_body,
        in_specs=[
            pl.BlockSpec((32, B), lambda: (0, 0)),
            pl.BlockSpec((EMB, 32), lambda: (0, 0)),
            pl.BlockSpec((RANK, 2, 4 * HID, EMB), lambda: (0, 0, 0, 0)),
            pl.BlockSpec((RANK, 2, 4 * HID, HID), lambda: (0, 0, 0, 0)),
            pl.BlockSpec((RANK, 2, 4 * HID, 8), lambda: (0, 0, 0, 0)),
            pl.BlockSpec((32, 2 * HID), lambda: (0, 0)),
            pl.BlockSpec((32, 8), lambda: (0, 0)),
        ],
        out_specs=pl.BlockSpec((RANK * STEPS, 32, B), lambda: (0, 0, 0)),
        out_shape=jax.ShapeDtypeStruct((RANK * STEPS, 32, B), jnp.float32),
    )(q_ohT, q_embT, q_Wih, q_Whh, qb, qlinP, qlbP)


@functools.partial(
    pl.kernel,
    out_type=(jax.ShapeDtypeStruct((NF_PAD,), jnp.int32),
              jax.ShapeDtypeStruct((NF_PAD,), jnp.float32)),
    mesh=_SC_MESH,
    compiler_params=_SC_PARAMS,
    scratch_types=[
        pltpu.VMEM((TPF * 3,), jnp.int32),
        pltpu.VMEM((TPF,), jnp.int32),
        pltpu.VMEM((TPF,), jnp.int32),
        pltpu.VMEM((TPF,), jnp.float32),
        pltpu.SemaphoreType.DMA,
    ],
)
def _pack_kernel(facts_hbm, ea_hbm, pk_hbm, vl_hbm, f_v, idx_v, pk_v, vl_v,
                 sem):
    wid = lax.axis_index("s") * 2 + lax.axis_index("c")
    base = wid * TPF
    pltpu.sync_copy(facts_hbm.at[pl.ds(base * 3, TPF * 3)], f_v)
    lanes = lax.iota(jnp.int32, 16)

    def grp(g, carry):
        row = (g * 16 + lanes) * 3
        rel = plsc.load_gather(f_v, [row])
        fh = plsc.load_gather(f_v, [row + 1])
        ft = plsc.load_gather(f_v, [row + 2])
        relc = jnp.minimum(rel, 12)
        pk_v[pl.ds(g * 16, 16)] = fh | (ft << 14) | (relc << 28)
        idx_v[pl.ds(g * 16, 16)] = fh * N_OPS + rel
        return carry

    lax.fori_loop(0, TPF // 16, grp, 0)
    pltpu.async_copy(ea_hbm.at[idx_v], vl_v, sem).wait()
    pltpu.sync_copy(pk_v, pk_hbm.at[pl.ds(base, TPF)])
    pltpu.sync_copy(vl_v, vl_hbm.at[pl.ds(base, TPF)])


@functools.partial(
    pl.kernel,
    out_type=jax.ShapeDtypeStruct((B, N_ENT), jnp.float32),
    mesh=_SC_MESH,
    compiler_params=_SC_PARAMS,
    scratch_types=[
        pltpu.VMEM((N_ENT,), jnp.float32),
        pltpu.VMEM((N_ENT,), jnp.float32),
        pltpu.VMEM((N_ENT,), jnp.float32),
        pltpu.VMEM((N_ENT,), jnp.float32),
        pltpu.VMEM((N_ENT,), jnp.float32),
        pltpu.VMEM((N_ENT,), jnp.float32),
        pltpu.VMEM((2 * CH,), jnp.int32),
        pltpu.VMEM((2 * CH,), jnp.float32),
        pltpu.VMEM((64,), jnp.float32),
        pltpu.VMEM((B,), jnp.int32),
        pltpu.SemaphoreType.DMA,
        pltpu.SemaphoreType.DMA,
    ],
)
def _prop_kernel(pk_hbm, vl_hbm, tbl_hbm, heads_hbm, out_hbm,
                 M0, M1, A0, A1, L0, L1, pk_v, vl_v, tbl_v, heads_v,
                 sem_pk, sem_vl):
    f32 = jnp.float32
    wid = lax.axis_index("s") * 2 + lax.axis_index("c")
    lanes = lax.iota(jnp.int32, 16)
    lane0 = lanes == 0
    NV = N_ENT // 16
    z16 = jnp.zeros(16, f32)
    ones16 = jnp.ones(16, f32)

    pltpu.sync_copy(heads_hbm, heads_v)
    h0v = plsc.load_gather(heads_v, [jnp.full((16,), 0, jnp.int32) + 2 * wid])
    h1v = plsc.load_gather(heads_v,
                           [jnp.full((16,), 1, jnp.int32) + 2 * wid])

    def fact_sweep(accs):
        pltpu.make_async_copy(pk_hbm.at[pl.ds(0, CH)],
                              pk_v.at[pl.ds(0, CH)], sem_pk).start()
        pltpu.make_async_copy(vl_hbm.at[pl.ds(0, CH)],
                              vl_v.at[pl.ds(0, CH)], sem_vl).start()

        def chunk(ci, a):
            par = lax.rem(ci, 2) * CH
            pltpu.make_async_copy(pk_hbm.at[pl.ds(0, CH)],
                                  pk_v.at[pl.ds(par, CH)], sem_pk).wait()
            pltpu.make_async_copy(vl_hbm.at[pl.ds(0, CH)],
                                  vl_v.at[pl.ds(par, CH)], sem_vl).wait()

            @pl.when(ci + 1 < NCH)
            def _():
                pltpu.make_async_copy(
                    pk_hbm.at[pl.ds((ci + 1) * CH, CH)],
                    pk_v.at[pl.ds(CH - par, CH)], sem_pk).start()
                pltpu.make_async_copy(
                    vl_hbm.at[pl.ds((ci + 1) * CH, CH)],
                    vl_v.at[pl.ds(CH - par, CH)], sem_vl).start()

            @plsc.parallel_loop(0, CH // 16, 1, unroll=5, carry=a)
            def grp(g, acc):
                a0, a1 = acc
                pk = pk_v[pl.ds(par + g * 16, 16)]
                vl = vl_v[pl.ds(par + g * 16, 16)]
                fh = pk & 0x3FFF
                ft = (pk >> 14) & 0x3FFF
                rl = (pk >> 28) & 0xF
                cf0 = vl * plsc.load_gather(tbl_v, [rl])
                cb0 = vl * plsc.load_gather(tbl_v, [rl + 16])
                cf1 = vl * plsc.load_gather(tbl_v, [rl + 32])
                cb1 = vl * plsc.load_gather(tbl_v, [rl + 48])
                m0h = plsc.load_gather(M0, [fh])
                m0t = plsc.load_gather(M0, [ft])
                m1h = plsc.load_gather(M1, [fh])
                m1t = plsc.load_gather(M1, [ft])
                p0f = m0h * cf0
                p0b = m0t * cb0
                p1f = m1h * cf1
                p1b = m1t * cb1
                plsc.addupdate_scatter(A0, [ft], p0f)
                plsc.addupdate_scatter(A0, [fh], p0b)
                plsc.addupdate_scatter(A1, [ft], p1f)
                plsc.addupdate_scatter(A1, [fh], p1b)
                return (a0 + (p0f + p0b), a1 + (p1f + p1b))

            return grp

        return lax.fori_loop(0, NCH, chunk, accs)

    def load_tbl(r, t):
        ti = r * STEPS + t
        pltpu.sync_copy(tbl_hbm.at[ti * NTILE + wid], tbl_v)
        a24_0 = plsc.load_gather(tbl_v, [jnp.full((16,), 15, jnp.int32)])
        a24_1 = plsc.load_gather(tbl_v, [jnp.full((16,), 47, jnp.int32)])
        return a24_0, a24_1

    for r in range(RANK):
        a24_0, a24_1 = load_tbl(r, 0)

        @plsc.parallel_loop(0, NV, 1, unroll=5)
        def zinit(i):
            M0[pl.ds(i * 16, 16)] = z16
            M1[pl.ds(i * 16, 16)] = z16
            A0[pl.ds(i * 16, 16)] = z16
            A1[pl.ds(i * 16, 16)] = z16

        plsc.store_scatter(M0, [h0v], ones16, mask=lane0)
        plsc.store_scatter(M1, [h1v], ones16, mask=lane0)
        plsc.store_scatter(A0, [h0v], a24_0, mask=lane0)
        plsc.store_scatter(A1, [h1v], a24_1, mask=lane0)
        sm0 = ones16
        sm1 = ones16

        for t in range(STEPS):
            acc0, acc1 = fact_sweep((z16, z16))

            nr0 = a24_0 * sm0 + jnp.broadcast_to(jnp.sum(acc0), (16,))
            nr1 = a24_1 * sm1 + jnp.broadcast_to(jnp.sum(acc1), (16,))
            rc0 = 1.0 / jnp.maximum(nr0, 1e-20)
            rc1 = 1.0 / jnp.maximum(nr1, 1e-20)
            sm0 = nr0 * rc0
            sm1 = nr1 * rc1

            if t < STEPS - 1:
                a24_0, a24_1 = load_tbl(r, t + 1)

                @plsc.parallel_loop(0, NV, 1, unroll=5)
                def renorm(i):
                    m0 = A0[pl.ds(i * 16, 16)] * rc0
                    m1 = A1[pl.ds(i * 16, 16)] * rc1
                    M0[pl.ds(i * 16, 16)] = m0
                    M1[pl.ds(i * 16, 16)] = m1
                    A0[pl.ds(i * 16, 16)] = a24_0 * m0
                    A1[pl.ds(i * 16, 16)] = a24_1 * m1
            elif r == 0:
                @plsc.parallel_loop(0, NV, 1, unroll=5)
                def linit(i):
                    L0[pl.ds(i * 16, 16)] = A0[pl.ds(i * 16, 16)] * rc0
                    L1[pl.ds(i * 16, 16)] = A1[pl.ds(i * 16, 16)] * rc1
            else:
                @plsc.parallel_loop(0, NV, 1, unroll=5)
                def lacc(i):
                    L0[pl.ds(i * 16, 16)] = (L0[pl.ds(i * 16, 16)]
                                             + A0[pl.ds(i * 16, 16)] * rc0)
                    L1[pl.ds(i * 16, 16)] = (L1[pl.ds(i * 16, 16)]
                                             + A1[pl.ds(i * 16, 16)] * rc1)

    pltpu.sync_copy(L0, out_hbm.at[2 * wid])
    pltpu.sync_copy(L1, out_hbm.at[2 * wid + 1])


def kernel(queries, heads, facts, entity_degrees, query_emb, entity_emb,
           q_Wih, q_Whh, q_bih, q_bhh, e_Wih, e_Whh, e_bih, e_bhh,
           qlin_W, qlin_b, elin_W, elin_b):
    f32 = jnp.float32

    emb_pad = jnp.zeros((32, EMB), f32).at[:N_OPS + 1].set(entity_emb)
    wihT = jnp.transpose(e_Wih, (0, 2, 1))
    whhT = jnp.transpose(e_Whh, (0, 2, 1))
    ebias = jnp.zeros((2, 8, 4 * HID), f32).at[:, 0, :].set(e_bih + e_bhh)
    elinT = jnp.transpose(elin_W)
    elbP = jnp.zeros((8, N_OPS), f32).at[0].set(elin_b)
    ea = _entity_lstm_call(entity_degrees.astype(jnp.int32), emb_pad, wihT,
                           whhT, ebias, elinT, elbP)

    q_ohT = jax.nn.one_hot(queries, 32, axis=0, dtype=f32)
    q_embT = jnp.zeros((EMB, 32), f32).at[:, :N_OPS].set(query_emb.T)
    qb = jnp.zeros((RANK, 2, 4 * HID, 8), f32).at[:, :, :, 0].set(
        q_bih + q_bhh)
    qlinP = jnp.zeros((32, 2 * HID), f32).at[:N_OPS + 1].set(qlin_W)
    qlbP = jnp.full((32, 8), -1e30, f32).at[:N_OPS + 1, :].set(
        qlin_b[:, None])
    attnT = _query_attn_call(q_ohT, q_embT, q_Wih, q_Whh, qb, qlinP, qlbP)

    fwd = attnT[:, 0:12, :]
    bwd = attnT[:, 12:24, :]
    a24 = attnT[:, 24:25, :]
    zero3 = jnp.zeros((RANK * STEPS, 3, B), f32)
    zero4 = jnp.zeros((RANK * STEPS, 4, B), f32)
    tbl = jnp.concatenate([fwd, zero3, a24, bwd, zero4], axis=1)
    tbl = jnp.transpose(tbl, (0, 2, 1)).reshape(RANK * STEPS * NTILE, 64)

    facts_pad = jnp.zeros((NF_PAD, 3), jnp.int32).at[:NF].set(
        facts.astype(jnp.int32))
    pk, vl = _pack_kernel(facts_pad.reshape(-1), ea.reshape(-1))

    return _prop_kernel(pk, vl, tbl, heads.astype(jnp.int32))


# speedup vs baseline: 76.0092x; 1.0311x over previous
import functools

import jax
import jax.numpy as jnp
from jax import lax
from jax.experimental import pallas as pl
from jax.experimental.pallas import tpu as pltpu
from jax.experimental.pallas import tpu_sc as plsc

N_ENT = 10000
N_OPS = 24
RANK = 3
STEPS = 3
B = 64
NF = 50000
DLEN = 8
EMB = 128
HID = 128

NTILE = 32
TPF = 1568
CH = 10000
NCH = NF // CH
EBLK = 1000

_SC_MESH = plsc.VectorSubcoreMesh(core_axis_name="c", subcore_axis_name="s")
_SC_PARAMS = pltpu.CompilerParams(needs_layout_passes=False)


def _entity_body(deg_ref, emb_ref, wih_ref, whh_ref, b_ref, elin_ref, elb_ref,
                 out_ref):
    f32 = jnp.float32
    deg = deg_ref[...]
    emb = emb_ref[...]

    xs = []
    for t in range(DLEN):
        oh = (deg[:, t:t + 1]
              == lax.broadcasted_iota(jnp.int32, (EBLK, 32), 1)).astype(f32)
        xs.append(jnp.dot(oh, emb, preferred_element_type=f32))

    dn = (((1,), (1,)), ((), ()))
    hs = []
    for d in range(2):
        wih = wih_ref[d]
        whh = whh_ref[d]
        bias = b_ref[d, 0:1, :]
        h = jnp.zeros((EBLK, HID), f32)
        c = jnp.zeros((EBLK, HID), f32)
        for s in range(DLEN):
            x = xs[s] if d == 0 else xs[DLEN - 1 - s]
            g = (lax.dot_general(x, wih, dn, preferred_element_type=f32)
                 + lax.dot_general(h, whh, dn, preferred_element_type=f32)
                 + bias)
            gi = jax.nn.sigmoid(g[:, 0:HID])
            gf = jax.nn.sigmoid(g[:, HID:2 * HID])
            gg = jnp.tanh(g[:, 2 * HID:3 * HID])
            go = jax.nn.sigmoid(g[:, 3 * HID:4 * HID])
            c = gf * c + gi * gg
            h = go * jnp.tanh(c)
        hs.append(h)
    hT = jnp.concatenate(hs, axis=1)
    logits = (lax.dot_general(hT, elin_ref[...], dn,
                              preferred_element_type=f32) + elb_ref[0:1, :])
    m = jnp.max(logits, axis=1, keepdims=True)
    e = jnp.exp(logits - m)
    out_ref[...] = e / jnp.sum(e, axis=1, keepdims=True)


def _entity_lstm_call(entity_degrees, emb_pad, wih, whh, ebias, elin, elbP):
    return pl.pallas_call(
        _entity_body,
        grid=(N_ENT // EBLK,),
        in_specs=[
            pl.BlockSpec((EBLK, DLEN), lambda i: (i, 0)),
            pl.BlockSpec((32, EMB), lambda i: (0, 0)),
            pl.BlockSpec((2, 4 * HID, EMB), lambda i: (0, 0, 0)),
            pl.BlockSpec((2, 4 * HID, HID), lambda i: (0, 0, 0)),
            pl.BlockSpec((2, 8, 4 * HID), lambda i: (0, 0, 0)),
            pl.BlockSpec((N_OPS, 2 * HID), lambda i: (0, 0)),
            pl.BlockSpec((8, N_OPS), lambda i: (0, 0)),
        ],
        out_specs=pl.BlockSpec((EBLK, N_OPS), lambda i: (i, 0)),
        out_shape=jax.ShapeDtypeStruct((N_ENT, N_OPS), jnp.float32),
    )(entity_degrees, emb_pad, wih, whh, ebias, elin, elbP)


def _query_body(ohT_ref, embT_ref, wih_ref, whh_ref, b_ref, qlin_ref, qlb_ref,
                out_ref):
    f32 = jnp.float32
    xT = jnp.dot(embT_ref[...], ohT_ref[...], preferred_element_type=f32)
    for r in range(RANK):
        hs = []
        for d in range(2):
            wih = wih_ref[r, d]
            whh = whh_ref[r, d]
            bias = b_ref[r, d, :, 0:1]
            h = jnp.zeros((HID, B), f32)
            c = jnp.zeros((HID, B), f32)
            hd = []
            for _ in range(STEPS):
                g = (jnp.dot(wih, xT, preferred_element_type=f32)
                     + jnp.dot(whh, h, preferred_element_type=f32) + bias)
                gi = jax.nn.sigmoid(g[0:HID])
                gf = jax.nn.sigmoid(g[HID:2 * HID])
                gg = jnp.tanh(g[2 * HID:3 * HID])
                go = jax.nn.sigmoid(g[3 * HID:4 * HID])
                c = gf * c + gi * gg
                h = go * jnp.tanh(c)
                hd.append(h)
            hs.append(hd)
        for t in range(STEPS):
            outT = jnp.concatenate([hs[0][t], hs[1][STEPS - 1 - t]], axis=0)
            lg = (jnp.dot(qlin_ref[...], outT, preferred_element_type=f32)
                  + qlb_ref[:, 0:1])
            m = jnp.max(lg, axis=0, keepdims=True)
            e = jnp.exp(lg - m)
            out_ref[r * STEPS + t] = e / jnp.sum(e, axis=0, keepdims=True)


def _query_attn_call(q_ohT, q_embT, q_Wih, q_Whh, qb, qlinP, qlbP):
    return pl.pallas_call(
        _query_body,
        in_specs=[
            pl.BlockSpec((32, B), lambda: (0, 0)),
            pl.BlockSpec((EMB, 32), lambda: (0, 0)),
            pl.BlockSpec((RANK, 2, 4 * HID, EMB), lambda: (0, 0, 0, 0)),
            pl.BlockSpec((RANK, 2, 4 * HID, HID), lambda: (0, 0, 0, 0)),
            pl.BlockSpec((RANK, 2, 4 * HID, 8), lambda: (0, 0, 0, 0)),
            pl.BlockSpec((32, 2 * HID), lambda: (0, 0)),
            pl.BlockSpec((32, 8), lambda: (0, 0)),
        ],
        out_specs=pl.BlockSpec((RANK * STEPS, 32, B), lambda: (0, 0, 0)),
        out_shape=jax.ShapeDtypeStruct((RANK * STEPS, 32, B), jnp.float32),
    )(q_ohT, q_embT, q_Wih, q_Whh, qb, qlinP, qlbP)


@functools.partial(
    pl.kernel,
    out_type=(jax.ShapeDtypeStruct((NF,), jnp.int32),
              jax.ShapeDtypeStruct((NF,), jnp.float32)),
    mesh=_SC_MESH,
    compiler_params=_SC_PARAMS,
    scratch_types=[
        pltpu.VMEM((TPF * 3,), jnp.int32),
        pltpu.VMEM((TPF,), jnp.int32),
        pltpu.VMEM((TPF,), jnp.int32),
        pltpu.VMEM((TPF,), jnp.float32),
        pltpu.SemaphoreType.DMA,
    ],
)
def _pack_kernel(facts_hbm, ea_hbm, pk_hbm, vl_hbm, f_v, idx_v, pk_v, vl_v,
                 sem):
    wid = lax.axis_index("s") * 2 + lax.axis_index("c")
    base = jnp.minimum(wid * TPF, NF - TPF)
    pltpu.sync_copy(facts_hbm.at[pl.ds(base * 3, TPF * 3)], f_v)
    lanes = lax.iota(jnp.int32, 16)

    def grp(g, carry):
        row = (g * 16 + lanes) * 3
        rel = plsc.load_gather(f_v, [row])
        fh = plsc.load_gather(f_v, [row + 1])
        ft = plsc.load_gather(f_v, [row + 2])
        relc = jnp.minimum(rel, 12)
        pk_v[pl.ds(g * 16, 16)] = fh | (ft << 14) | (relc << 28)
        idx_v[pl.ds(g * 16, 16)] = fh * N_OPS + rel
        return carry

    lax.fori_loop(0, TPF // 16, grp, 0)
    pltpu.async_copy(ea_hbm.at[idx_v], vl_v, sem).wait()
    pltpu.sync_copy(pk_v, pk_hbm.at[pl.ds(base, TPF)])
    pltpu.sync_copy(vl_v, vl_hbm.at[pl.ds(base, TPF)])


@functools.partial(
    pl.kernel,
    out_type=jax.ShapeDtypeStruct((B, N_ENT), jnp.float32),
    mesh=_SC_MESH,
    compiler_params=_SC_PARAMS,
    scratch_types=[
        pltpu.VMEM((N_ENT,), jnp.float32),
        pltpu.VMEM((N_ENT,), jnp.float32),
        pltpu.VMEM((N_ENT,), jnp.float32),
        pltpu.VMEM((N_ENT,), jnp.float32),
        pltpu.VMEM((N_ENT,), jnp.float32),
        pltpu.VMEM((N_ENT,), jnp.float32),
        pltpu.VMEM((2 * CH,), jnp.int32),
        pltpu.VMEM((2 * CH,), jnp.float32),
        pltpu.VMEM((64,), jnp.float32),
        pltpu.VMEM((B,), jnp.int32),
        pltpu.SemaphoreType.DMA,
        pltpu.SemaphoreType.DMA,
    ],
)
def _prop_kernel(pk_hbm, vl_hbm, tbl_hbm, heads_hbm, out_hbm,
                 M0, M1, A0, A1, L0, L1, pk_v, vl_v, tbl_v, heads_v,
                 sem_pk, sem_vl):
    f32 = jnp.float32
    wid = lax.axis_index("s") * 2 + lax.axis_index("c")
    lanes = lax.iota(jnp.int32, 16)
    lane0 = lanes == 0
    NV = N_ENT // 16
    z16 = jnp.zeros(16, f32)
    ones16 = jnp.ones(16, f32)

    pltpu.sync_copy(heads_hbm, heads_v)
    h0v = plsc.load_gather(heads_v, [jnp.full((16,), 0, jnp.int32) + 2 * wid])
    h1v = plsc.load_gather(heads_v,
                           [jnp.full((16,), 1, jnp.int32) + 2 * wid])

    def fact_sweep(accs):
        pltpu.make_async_copy(pk_hbm.at[pl.ds(0, CH)],
                              pk_v.at[pl.ds(0, CH)], sem_pk).start()
        pltpu.make_async_copy(vl_hbm.at[pl.ds(0, CH)],
                              vl_v.at[pl.ds(0, CH)], sem_vl).start()

        def chunk(ci, a):
            par = lax.rem(ci, 2) * CH
            pltpu.make_async_copy(pk_hbm.at[pl.ds(0, CH)],
                                  pk_v.at[pl.ds(par, CH)], sem_pk).wait()
            pltpu.make_async_copy(vl_hbm.at[pl.ds(0, CH)],
                                  vl_v.at[pl.ds(par, CH)], sem_vl).wait()

            @pl.when(ci + 1 < NCH)
            def _():
                pltpu.make_async_copy(
                    pk_hbm.at[pl.ds((ci + 1) * CH, CH)],
                    pk_v.at[pl.ds(CH - par, CH)], sem_pk).start()
                pltpu.make_async_copy(
                    vl_hbm.at[pl.ds((ci + 1) * CH, CH)],
                    vl_v.at[pl.ds(CH - par, CH)], sem_vl).start()

            @plsc.parallel_loop(0, CH // 16, 1, unroll=5, carry=a)
            def grp(g, acc):
                a0, a1 = acc
                pk = pk_v[pl.ds(par + g * 16, 16)]
                vl = vl_v[pl.ds(par + g * 16, 16)]
                fh = pk & 0x3FFF
                ft = (pk >> 14) & 0x3FFF
                rl = (pk >> 28) & 0xF
                cf0 = vl * plsc.load_gather(tbl_v, [rl])
                cb0 = vl * plsc.load_gather(tbl_v, [rl + 16])
                cf1 = vl * plsc.load_gather(tbl_v, [rl + 32])
                cb1 = vl * plsc.load_gather(tbl_v, [rl + 48])
                m0h = plsc.load_gather(M0, [fh])
                m0t = plsc.load_gather(M0, [ft])
                m1h = plsc.load_gather(M1, [fh])
                m1t = plsc.load_gather(M1, [ft])
                p0f = m0h * cf0
                p0b = m0t * cb0
                p1f = m1h * cf1
                p1b = m1t * cb1
                plsc.addupdate_scatter(A0, [ft], p0f)
                plsc.addupdate_scatter(A0, [fh], p0b)
                plsc.addupdate_scatter(A1, [ft], p1f)
                plsc.addupdate_scatter(A1, [fh], p1b)
                return (a0 + (p0f + p0b), a1 + (p1f + p1b))

            return grp

        return lax.fori_loop(0, NCH, chunk, accs)

    def load_tbl(r, t):
        ti = r * STEPS + t
        pltpu.sync_copy(tbl_hbm.at[ti * NTILE + wid], tbl_v)
        a24_0 = plsc.load_gather(tbl_v, [jnp.full((16,), 15, jnp.int32)])
        a24_1 = plsc.load_gather(tbl_v, [jnp.full((16,), 47, jnp.int32)])
        return a24_0, a24_1

    for r in range(RANK):
        a24_0, a24_1 = load_tbl(r, 0)

        @plsc.parallel_loop(0, NV, 1, unroll=5)
        def zinit(i):
            M0[pl.ds(i * 16, 16)] = z16
            M1[pl.ds(i * 16, 16)] = z16
            A0[pl.ds(i * 16, 16)] = z16
            A1[pl.ds(i * 16, 16)] = z16

        plsc.store_scatter(M0, [h0v], ones16, mask=lane0)
        plsc.store_scatter(M1, [h1v], ones16, mask=lane0)
        plsc.store_scatter(A0, [h0v], a24_0, mask=lane0)
        plsc.store_scatter(A1, [h1v], a24_1, mask=lane0)
        sm0 = ones16
        sm1 = ones16

        for t in range(STEPS):
            acc0, acc1 = fact_sweep((z16, z16))

            nr0 = a24_0 * sm0 + jnp.broadcast_to(jnp.sum(acc0), (16,))
            nr1 = a24_1 * sm1 + jnp.broadcast_to(jnp.sum(acc1), (16,))
            rc0 = 1.0 / jnp.maximum(nr0, 1e-20)
            rc1 = 1.0 / jnp.maximum(nr1, 1e-20)
            sm0 = nr0 * rc0
            sm1 = nr1 * rc1

            if t < STEPS - 1:
                a24_0, a24_1 = load_tbl(r, t + 1)

                @plsc.parallel_loop(0, NV, 1, unroll=5)
                def renorm(i):
                    m0 = A0[pl.ds(i * 16, 16)] * rc0
                    m1 = A1[pl.ds(i * 16, 16)] * rc1
                    M0[pl.ds(i * 16, 16)] = m0
                    M1[pl.ds(i * 16, 16)] = m1
                    A0[pl.ds(i * 16, 16)] = a24_0 * m0
                    A1[pl.ds(i * 16, 16)] = a24_1 * m1
            elif r == 0:
                @plsc.parallel_loop(0, NV, 1, unroll=5)
                def linit(i):
                    L0[pl.ds(i * 16, 16)] = A0[pl.ds(i * 16, 16)] * rc0
                    L1[pl.ds(i * 16, 16)] = A1[pl.ds(i * 16, 16)] * rc1
            else:
                @plsc.parallel_loop(0, NV, 1, unroll=5)
                def lacc(i):
                    L0[pl.ds(i * 16, 16)] = (L0[pl.ds(i * 16, 16)]
                                             + A0[pl.ds(i * 16, 16)] * rc0)
                    L1[pl.ds(i * 16, 16)] = (L1[pl.ds(i * 16, 16)]
                                             + A1[pl.ds(i * 16, 16)] * rc1)

    pltpu.sync_copy(L0, out_hbm.at[2 * wid])
    pltpu.sync_copy(L1, out_hbm.at[2 * wid + 1])


def kernel(queries, heads, facts, entity_degrees, query_emb, entity_emb,
           q_Wih, q_Whh, q_bih, q_bhh, e_Wih, e_Whh, e_bih, e_bhh,
           qlin_W, qlin_b, elin_W, elin_b):
    f32 = jnp.float32

    emb_pad = jnp.zeros((32, EMB), f32).at[:N_OPS + 1].set(entity_emb)
    ebias = jnp.zeros((2, 8, 4 * HID), f32).at[:, 0, :].set(e_bih + e_bhh)
    elbP = jnp.zeros((8, N_OPS), f32).at[0].set(elin_b)
    ea = _entity_lstm_call(entity_degrees.astype(jnp.int32), emb_pad, e_Wih,
                           e_Whh, ebias, elin_W, elbP)

    q_ohT = jax.nn.one_hot(queries, 32, axis=0, dtype=f32)
    q_embT = jnp.zeros((EMB, 32), f32).at[:, :N_OPS].set(query_emb.T)
    qb = jnp.zeros((RANK, 2, 4 * HID, 8), f32).at[:, :, :, 0].set(
        q_bih + q_bhh)
    qlinP = jnp.zeros((32, 2 * HID), f32).at[:N_OPS + 1].set(qlin_W)
    qlbP = jnp.full((32, 8), -1e30, f32).at[:N_OPS + 1, :].set(
        qlin_b[:, None])
    attnT = _query_attn_call(q_ohT, q_embT, q_Wih, q_Whh, qb, qlinP, qlbP)

    fwd = attnT[:, 0:12, :]
    bwd = attnT[:, 12:24, :]
    a24 = attnT[:, 24:25, :]
    zero3 = jnp.zeros((RANK * STEPS, 3, B), f32)
    zero4 = jnp.zeros((RANK * STEPS, 4, B), f32)
    tbl = jnp.concatenate([fwd, zero3, a24, bwd, zero4], axis=1)
    tbl = jnp.transpose(tbl, (0, 2, 1)).reshape(RANK * STEPS * NTILE, 64)

    pk, vl = _pack_kernel(facts.astype(jnp.int32).reshape(-1),
                          ea.reshape(-1))

    return _prop_kernel(pk, vl, tbl, heads.astype(jnp.int32))


# speedup vs baseline: 80.3286x; 1.0568x over previous
import functools

import jax
import jax.numpy as jnp
from jax import lax
from jax.experimental import pallas as pl
from jax.experimental.pallas import tpu as pltpu
from jax.experimental.pallas import tpu_sc as plsc

N_ENT = 10000
N_OPS = 24
RANK = 3
STEPS = 3
B = 64
NF = 50000
DLEN = 8
EMB = 128
HID = 128

NTILE = 32
TPF = 1568
CH = 10000
NCH = NF // CH
EBLK = 1000

_SC_MESH = plsc.VectorSubcoreMesh(core_axis_name="c", subcore_axis_name="s")
_SC_PARAMS = pltpu.CompilerParams(needs_layout_passes=False)


def _entity_body(deg_ref, emb_ref, wih_ref, whh_ref, b_ref, elin_ref, elb_ref,
                 out_ref):
    f32 = jnp.float32
    deg = deg_ref[...]
    emb = emb_ref[...]

    xs = []
    for t in range(DLEN):
        oh = (deg[:, t:t + 1]
              == lax.broadcasted_iota(jnp.int32, (EBLK, 32), 1)).astype(f32)
        xs.append(jnp.dot(oh, emb, preferred_element_type=f32))

    dn = (((1,), (1,)), ((), ()))
    hs = []
    for d in range(2):
        wih = wih_ref[d]
        whh = whh_ref[d]
        bias = b_ref[d, 0:1, :]
        h = jnp.zeros((EBLK, HID), f32)
        c = jnp.zeros((EBLK, HID), f32)
        for s in range(DLEN):
            x = xs[s] if d == 0 else xs[DLEN - 1 - s]
            g = (lax.dot_general(x, wih, dn, preferred_element_type=f32)
                 + lax.dot_general(h, whh, dn, preferred_element_type=f32)
                 + bias)
            gi = jax.nn.sigmoid(g[:, 0:HID])
            gf = jax.nn.sigmoid(g[:, HID:2 * HID])
            gg = jnp.tanh(g[:, 2 * HID:3 * HID])
            go = jax.nn.sigmoid(g[:, 3 * HID:4 * HID])
            c = gf * c + gi * gg
            h = go * jnp.tanh(c)
        hs.append(h)
    hT = jnp.concatenate(hs, axis=1)
    logits = (lax.dot_general(hT, elin_ref[...], dn,
                              preferred_element_type=f32) + elb_ref[0:1, :])
    m = jnp.max(logits, axis=1, keepdims=True)
    e = jnp.exp(logits - m)
    out_ref[...] = e / jnp.sum(e, axis=1, keepdims=True)


def _entity_lstm_call(entity_degrees, emb_pad, wih, whh, ebias, elin, elbP):
    return pl.pallas_call(
        _entity_body,
        grid=(N_ENT // EBLK,),
        in_specs=[
            pl.BlockSpec((EBLK, DLEN), lambda i: (i, 0)),
            pl.BlockSpec((32, EMB), lambda i: (0, 0)),
            pl.BlockSpec((2, 4 * HID, EMB), lambda i: (0, 0, 0)),
            pl.BlockSpec((2, 4 * HID, HID), lambda i: (0, 0, 0)),
            pl.BlockSpec((2, 8, 4 * HID), lambda i: (0, 0, 0)),
            pl.BlockSpec((N_OPS, 2 * HID), lambda i: (0, 0)),
            pl.BlockSpec((8, N_OPS), lambda i: (0, 0)),
        ],
        out_specs=pl.BlockSpec((EBLK, N_OPS), lambda i: (i, 0)),
        out_shape=jax.ShapeDtypeStruct((N_ENT, N_OPS), jnp.float32),
    )(entity_degrees, emb_pad, wih, whh, ebias, elin, elbP)


def _query_body(ohT_ref, embT_ref, wih_ref, whh_ref, b_ref, qlin_ref, qlb_ref,
                out_ref):
    f32 = jnp.float32
    xT = jnp.dot(embT_ref[...], ohT_ref[...], preferred_element_type=f32)
    for r in range(RANK):
        hs = []
        for d in range(2):
            wih = wih_ref[r, d]
            whh = whh_ref[r, d]
            bias = b_ref[r, d, :, 0:1]
            h = jnp.zeros((HID, B), f32)
            c = jnp.zeros((HID, B), f32)
            hd = []
            for _ in range(STEPS):
                g = (jnp.dot(wih, xT, preferred_element_type=f32)
                     + jnp.dot(whh, h, preferred_element_type=f32) + bias)
                gi = jax.nn.sigmoid(g[0:HID])
                gf = jax.nn.sigmoid(g[HID:2 * HID])
                gg = jnp.tanh(g[2 * HID:3 * HID])
                go = jax.nn.sigmoid(g[3 * HID:4 * HID])
                c = gf * c + gi * gg
                h = go * jnp.tanh(c)
                hd.append(h)
            hs.append(hd)
        for t in range(STEPS):
            outT = jnp.concatenate([hs[0][t], hs[1][STEPS - 1 - t]], axis=0)
            lg = (jnp.dot(qlin_ref[...], outT, preferred_element_type=f32)
                  + qlb_ref[:, 0:1])
            m = jnp.max(lg, axis=0, keepdims=True)
            e = jnp.exp(lg - m)
            out_ref[r * STEPS + t] = e / jnp.sum(e, axis=0, keepdims=True)


def _query_attn_call(q_ohT, q_embT, q_Wih, q_Whh, qb, qlinP, qlbP):
    return pl.pallas_call(
        _query_body,
        in_specs=[
            pl.BlockSpec((32, B), lambda: (0, 0)),
            pl.BlockSpec((EMB, 32), lambda: (0, 0)),
            pl.BlockSpec((RANK, 2, 4 * HID, EMB), lambda: (0, 0, 0, 0)),
            pl.BlockSpec((RANK, 2, 4 * HID, HID), lambda: (0, 0, 0, 0)),
            pl.BlockSpec((RANK, 2, 4 * HID, 8), lambda: (0, 0, 0, 0)),
            pl.BlockSpec((32, 2 * HID), lambda: (0, 0)),
            pl.BlockSpec((32, 8), lambda: (0, 0)),
        ],
        out_specs=pl.BlockSpec((RANK * STEPS, 32, B), lambda: (0, 0, 0)),
        out_shape=jax.ShapeDtypeStruct((RANK * STEPS, 32, B), jnp.float32),
    )(q_ohT, q_embT, q_Wih, q_Whh, qb, qlinP, qlbP)


_DUMMY_PK = -1073741824


@functools.partial(
    pl.kernel,
    out_type=(jax.ShapeDtypeStruct((NTILE * TPF,), jnp.int32),
              jax.ShapeDtypeStruct((NTILE * TPF,), jnp.float32),
              jax.ShapeDtypeStruct((NTILE * 16,), jnp.int32)),
    mesh=_SC_MESH,
    compiler_params=_SC_PARAMS,
    scratch_types=[
        pltpu.VMEM((TPF * 3,), jnp.int32),
        pltpu.VMEM((TPF + 16,), jnp.int32),
        pltpu.VMEM((TPF + 16,), jnp.int32),
        pltpu.VMEM((TPF,), jnp.float32),
        pltpu.VMEM((16,), jnp.int32),
        pltpu.SemaphoreType.DMA,
    ],
)
def _pack_kernel(facts_hbm, ea_hbm, pk_hbm, vl_hbm, cnt_hbm,
                 f_v, idx_v, pk_v, vl_v, cnt_v, sem):
    wid = lax.axis_index("s") * 2 + lax.axis_index("c")
    true_start = wid * TPF
    true_end = jnp.minimum(true_start + TPF, NF)
    buf_base = jnp.minimum(true_start, NF - TPF)
    g0 = (true_start - buf_base) // 16
    g1 = g0 + (true_end - true_start) // 16
    pltpu.sync_copy(facts_hbm.at[pl.ds(buf_base * 3, TPF * 3)], f_v)
    lanes = lax.iota(jnp.int32, 16)
    dummy16 = jnp.full((16,), _DUMMY_PK, jnp.int32)
    zi16 = jnp.zeros(16, jnp.int32)

    @plsc.parallel_loop(0, TPF // 16 + 1, 1, unroll=3)
    def prefill(g):
        pk_v[pl.ds(g * 16, 16)] = dummy16
        idx_v[pl.ds(g * 16, 16)] = zi16

    def grp(g, off):
        row = (g * 16 + lanes) * 3
        rel = plsc.load_gather(f_v, [row])
        fh = plsc.load_gather(f_v, [row + 1])
        ft = plsc.load_gather(f_v, [row + 2])
        mask = rel < 12
        pk = fh | (ft << 14) | (rel << 28)
        plsc.store_compressed(pk_v.at[pl.ds(off, 16)], pk, mask=mask)
        plsc.store_compressed(idx_v.at[pl.ds(off, 16)], fh * N_OPS + rel,
                              mask=mask)
        return off + jnp.sum(mask.astype(jnp.int32))

    off = lax.fori_loop(g0, g1, grp, jnp.int32(0))
    ng = (off + 15) // 16
    cnt_v[...] = jnp.broadcast_to(ng, (16,))
    pltpu.async_copy(ea_hbm.at[idx_v.at[pl.ds(0, TPF)]], vl_v, sem).wait()
    pltpu.sync_copy(pk_v.at[pl.ds(0, TPF)], pk_hbm.at[pl.ds(wid * TPF, TPF)])
    pltpu.sync_copy(vl_v, vl_hbm.at[pl.ds(wid * TPF, TPF)])
    pltpu.sync_copy(cnt_v, cnt_hbm.at[pl.ds(wid * 16, 16)])


@functools.partial(
    pl.kernel,
    out_type=jax.ShapeDtypeStruct((B, N_ENT), jnp.float32),
    mesh=_SC_MESH,
    compiler_params=_SC_PARAMS,
    scratch_types=[
        pltpu.VMEM((N_ENT,), jnp.float32),
        pltpu.VMEM((N_ENT,), jnp.float32),
        pltpu.VMEM((N_ENT,), jnp.float32),
        pltpu.VMEM((N_ENT,), jnp.float32),
        pltpu.VMEM((N_ENT,), jnp.float32),
        pltpu.VMEM((N_ENT,), jnp.float32),
        pltpu.VMEM((2 * TPF,), jnp.int32),
        pltpu.VMEM((2 * TPF,), jnp.float32),
        pltpu.VMEM((64,), jnp.float32),
        pltpu.VMEM((B,), jnp.int32),
        pltpu.VMEM((NTILE * 16,), jnp.int32),
        pltpu.SemaphoreType.DMA,
        pltpu.SemaphoreType.DMA,
    ],
)
def _prop_kernel(pk_hbm, vl_hbm, cnt_hbm, tbl_hbm, heads_hbm, out_hbm,
                 M0, M1, A0, A1, L0, L1, pk_v, vl_v, tbl_v, heads_v, cnt_v,
                 sem_pk, sem_vl):
    f32 = jnp.float32
    wid = lax.axis_index("s") * 2 + lax.axis_index("c")
    lanes = lax.iota(jnp.int32, 16)
    lane0 = lanes == 0
    NV = N_ENT // 16
    z16 = jnp.zeros(16, f32)
    ones16 = jnp.ones(16, f32)

    pltpu.sync_copy(heads_hbm, heads_v)
    pltpu.sync_copy(cnt_hbm, cnt_v)
    h0v = plsc.load_gather(heads_v, [jnp.full((16,), 0, jnp.int32) + 2 * wid])
    h1v = plsc.load_gather(heads_v,
                           [jnp.full((16,), 1, jnp.int32) + 2 * wid])

    def fact_sweep(accs):
        pltpu.make_async_copy(pk_hbm.at[pl.ds(0, TPF)],
                              pk_v.at[pl.ds(0, TPF)], sem_pk).start()
        pltpu.make_async_copy(vl_hbm.at[pl.ds(0, TPF)],
                              vl_v.at[pl.ds(0, TPF)], sem_vl).start()

        def chunk(ci, a):
            par = lax.rem(ci, 2) * TPF
            ng = cnt_v[pl.ds(ci * 16, 16)][0]
            pltpu.make_async_copy(pk_hbm.at[pl.ds(0, TPF)],
                                  pk_v.at[pl.ds(par, TPF)], sem_pk).wait()
            pltpu.make_async_copy(vl_hbm.at[pl.ds(0, TPF)],
                                  vl_v.at[pl.ds(par, TPF)], sem_vl).wait()

            @pl.when(ci + 1 < NTILE)
            def _():
                pltpu.make_async_copy(
                    pk_hbm.at[pl.ds((ci + 1) * TPF, TPF)],
                    pk_v.at[pl.ds(TPF - par, TPF)], sem_pk).start()
                pltpu.make_async_copy(
                    vl_hbm.at[pl.ds((ci + 1) * TPF, TPF)],
                    vl_v.at[pl.ds(TPF - par, TPF)], sem_vl).start()

            @plsc.parallel_loop(0, ng, 1, unroll=2, carry=a)
            def grp(g, acc):
                a0, a1 = acc
                pk = pk_v[pl.ds(par + g * 16, 16)]
                vl = vl_v[pl.ds(par + g * 16, 16)]
                fh = pk & 0x3FFF
                ft = (pk >> 14) & 0x3FFF
                rl = (pk >> 28) & 0xF
                cf0 = vl * plsc.load_gather(tbl_v, [rl])
                cb0 = vl * plsc.load_gather(tbl_v, [rl + 16])
                cf1 = vl * plsc.load_gather(tbl_v, [rl + 32])
                cb1 = vl * plsc.load_gather(tbl_v, [rl + 48])
                m0h = plsc.load_gather(M0, [fh])
                m0t = plsc.load_gather(M0, [ft])
                m1h = plsc.load_gather(M1, [fh])
                m1t = plsc.load_gather(M1, [ft])
                p0f = m0h * cf0
                p0b = m0t * cb0
                p1f = m1h * cf1
                p1b = m1t * cb1
                plsc.addupdate_scatter(A0, [ft], p0f)
                plsc.addupdate_scatter(A0, [fh], p0b)
                plsc.addupdate_scatter(A1, [ft], p1f)
                plsc.addupdate_scatter(A1, [fh], p1b)
                return (a0 + (p0f + p0b), a1 + (p1f + p1b))

            return grp

        return lax.fori_loop(0, NTILE, chunk, accs)

    def load_tbl(r, t):
        ti = r * STEPS + t
        pltpu.sync_copy(tbl_hbm.at[ti * NTILE + wid], tbl_v)
        a24_0 = plsc.load_gather(tbl_v, [jnp.full((16,), 15, jnp.int32)])
        a24_1 = plsc.load_gather(tbl_v, [jnp.full((16,), 47, jnp.int32)])
        return a24_0, a24_1

    for r in range(RANK):
        a24_0, a24_1 = load_tbl(r, 0)

        @plsc.parallel_loop(0, NV, 1, unroll=5)
        def zinit(i):
            M0[pl.ds(i * 16, 16)] = z16
            M1[pl.ds(i * 16, 16)] = z16
            A0[pl.ds(i * 16, 16)] = z16
            A1[pl.ds(i * 16, 16)] = z16

        plsc.store_scatter(M0, [h0v], ones16, mask=lane0)
        plsc.store_scatter(M1, [h1v], ones16, mask=lane0)
        plsc.store_scatter(A0, [h0v], a24_0, mask=lane0)
        plsc.store_scatter(A1, [h1v], a24_1, mask=lane0)
        sm0 = ones16
        sm1 = ones16

        for t in range(STEPS):
            acc0, acc1 = fact_sweep((z16, z16))

            nr0 = a24_0 * sm0 + jnp.broadcast_to(jnp.sum(acc0), (16,))
            nr1 = a24_1 * sm1 + jnp.broadcast_to(jnp.sum(acc1), (16,))
            rc0 = 1.0 / jnp.maximum(nr0, 1e-20)
            rc1 = 1.0 / jnp.maximum(nr1, 1e-20)
            sm0 = nr0 * rc0
            sm1 = nr1 * rc1

            if t < STEPS - 1:
                a24_0, a24_1 = load_tbl(r, t + 1)

                @plsc.parallel_loop(0, NV, 1, unroll=5)
                def renorm(i):
                    m0 = A0[pl.ds(i * 16, 16)] * rc0
                    m1 = A1[pl.ds(i * 16, 16)] * rc1
                    M0[pl.ds(i * 16, 16)] = m0
                    M1[pl.ds(i * 16, 16)] = m1
                    A0[pl.ds(i * 16, 16)] = a24_0 * m0
                    A1[pl.ds(i * 16, 16)] = a24_1 * m1
            elif r == 0:
                @plsc.parallel_loop(0, NV, 1, unroll=5)
                def linit(i):
                    L0[pl.ds(i * 16, 16)] = A0[pl.ds(i * 16, 16)] * rc0
                    L1[pl.ds(i * 16, 16)] = A1[pl.ds(i * 16, 16)] * rc1
            else:
                @plsc.parallel_loop(0, NV, 1, unroll=5)
                def lacc(i):
                    L0[pl.ds(i * 16, 16)] = (L0[pl.ds(i * 16, 16)]
                                             + A0[pl.ds(i * 16, 16)] * rc0)
                    L1[pl.ds(i * 16, 16)] = (L1[pl.ds(i * 16, 16)]
                                             + A1[pl.ds(i * 16, 16)] * rc1)

    pltpu.sync_copy(L0, out_hbm.at[2 * wid])
    pltpu.sync_copy(L1, out_hbm.at[2 * wid + 1])


def kernel(queries, heads, facts, entity_degrees, query_emb, entity_emb,
           q_Wih, q_Whh, q_bih, q_bhh, e_Wih, e_Whh, e_bih, e_bhh,
           qlin_W, qlin_b, elin_W, elin_b):
    f32 = jnp.float32

    emb_pad = jnp.zeros((32, EMB), f32).at[:N_OPS + 1].set(entity_emb)
    ebias = jnp.zeros((2, 8, 4 * HID), f32).at[:, 0, :].set(e_bih + e_bhh)
    elbP = jnp.zeros((8, N_OPS), f32).at[0].set(elin_b)
    ea = _entity_lstm_call(entity_degrees.astype(jnp.int32), emb_pad, e_Wih,
                           e_Whh, ebias, elin_W, elbP)

    q_ohT = jax.nn.one_hot(queries, 32, axis=0, dtype=f32)
    q_embT = jnp.zeros((EMB, 32), f32).at[:, :N_OPS].set(query_emb.T)
    qb = jnp.zeros((RANK, 2, 4 * HID, 8), f32).at[:, :, :, 0].set(
        q_bih + q_bhh)
    qlinP = jnp.zeros((32, 2 * HID), f32).at[:N_OPS + 1].set(qlin_W)
    qlbP = jnp.full((32, 8), -1e30, f32).at[:N_OPS + 1, :].set(
        qlin_b[:, None])
    attnT = _query_attn_call(q_ohT, q_embT, q_Wih, q_Whh, qb, qlinP, qlbP)

    fwd = attnT[:, 0:12, :]
    bwd = attnT[:, 12:24, :]
    a24 = attnT[:, 24:25, :]
    zero3 = jnp.zeros((RANK * STEPS, 3, B), f32)
    zero4 = jnp.zeros((RANK * STEPS, 4, B), f32)
    tbl = jnp.concatenate([fwd, zero3, a24, bwd, zero4], axis=1)
    tbl = jnp.transpose(tbl, (0, 2, 1)).reshape(RANK * STEPS * NTILE, 64)

    pk, vl, cnt = _pack_kernel(facts.astype(jnp.int32).reshape(-1),
                               ea.reshape(-1))

    return _prop_kernel(pk, vl, cnt, tbl, heads.astype(jnp.int32))


# speedup vs baseline: 81.9666x; 1.0204x over previous
import functools

import jax
import jax.numpy as jnp
from jax import lax
from jax.experimental import pallas as pl
from jax.experimental.pallas import tpu as pltpu
from jax.experimental.pallas import tpu_sc as plsc

N_ENT = 10000
N_OPS = 24
RANK = 3
STEPS = 3
B = 64
NF = 50000
DLEN = 8
EMB = 128
HID = 128

NTILE = 32
TPF = 1568
CH = 10000
NCH = NF // CH
EBLK = 1000

_SC_MESH = plsc.VectorSubcoreMesh(core_axis_name="c", subcore_axis_name="s")
_SC_PARAMS = pltpu.CompilerParams(needs_layout_passes=False)


def _entity_body(deg_ref, emb_ref, wih_ref, whh_ref, b_ref, elin_ref, elb_ref,
                 out_ref):
    f32 = jnp.float32
    deg = deg_ref[...]
    emb = emb_ref[...]

    xs = []
    for t in range(DLEN):
        oh = (deg[:, t:t + 1]
              == lax.broadcasted_iota(jnp.int32, (EBLK, 32), 1)).astype(f32)
        xs.append(jnp.dot(oh, emb, preferred_element_type=f32))

    dn = (((1,), (1,)), ((), ()))
    hs = []
    for d in range(2):
        wih = wih_ref[d]
        whh = whh_ref[d]
        bias = b_ref[d, 0:1, :]
        h = jnp.zeros((EBLK, HID), f32)
        c = jnp.zeros((EBLK, HID), f32)
        for s in range(DLEN):
            x = xs[s] if d == 0 else xs[DLEN - 1 - s]
            g = (lax.dot_general(x, wih, dn, preferred_element_type=f32)
                 + lax.dot_general(h, whh, dn, preferred_element_type=f32)
                 + bias)
            gi = jax.nn.sigmoid(g[:, 0:HID])
            gf = jax.nn.sigmoid(g[:, HID:2 * HID])
            gg = jnp.tanh(g[:, 2 * HID:3 * HID])
            go = jax.nn.sigmoid(g[:, 3 * HID:4 * HID])
            c = gf * c + gi * gg
            h = go * jnp.tanh(c)
        hs.append(h)
    hT = jnp.concatenate(hs, axis=1)
    logits = (lax.dot_general(hT, elin_ref[...], dn,
                              preferred_element_type=f32) + elb_ref[0:1, :])
    m = jnp.max(logits, axis=1, keepdims=True)
    e = jnp.exp(logits - m)
    out_ref[...] = e / jnp.sum(e, axis=1, keepdims=True)


def _entity_lstm_call(entity_degrees, emb_pad, wih, whh, ebias, elin, elbP):
    return pl.pallas_call(
        _entity_body,
        grid=(N_ENT // EBLK,),
        in_specs=[
            pl.BlockSpec((EBLK, DLEN), lambda i: (i, 0)),
            pl.BlockSpec((32, EMB), lambda i: (0, 0)),
            pl.BlockSpec((2, 4 * HID, EMB), lambda i: (0, 0, 0)),
            pl.BlockSpec((2, 4 * HID, HID), lambda i: (0, 0, 0)),
            pl.BlockSpec((2, 8, 4 * HID), lambda i: (0, 0, 0)),
            pl.BlockSpec((N_OPS, 2 * HID), lambda i: (0, 0)),
            pl.BlockSpec((8, N_OPS), lambda i: (0, 0)),
        ],
        out_specs=pl.BlockSpec((EBLK, N_OPS), lambda i: (i, 0)),
        out_shape=jax.ShapeDtypeStruct((N_ENT, N_OPS), jnp.float32),
    )(entity_degrees, emb_pad, wih, whh, ebias, elin, elbP)


def _query_body(ohT_ref, embT_ref, wih_ref, whh_ref, b_ref, qlin_ref, qlb_ref,
                out_ref):
    f32 = jnp.float32
    xT = jnp.dot(embT_ref[...], ohT_ref[...], preferred_element_type=f32)
    for r in range(RANK):
        hs = []
        for d in range(2):
            wih = wih_ref[r, d]
            whh = whh_ref[r, d]
            bias = b_ref[r, d, :, 0:1]
            h = jnp.zeros((HID, B), f32)
            c = jnp.zeros((HID, B), f32)
            hd = []
            for _ in range(STEPS):
                g = (jnp.dot(wih, xT, preferred_element_type=f32)
                     + jnp.dot(whh, h, preferred_element_type=f32) + bias)
                gi = jax.nn.sigmoid(g[0:HID])
                gf = jax.nn.sigmoid(g[HID:2 * HID])
                gg = jnp.tanh(g[2 * HID:3 * HID])
                go = jax.nn.sigmoid(g[3 * HID:4 * HID])
                c = gf * c + gi * gg
                h = go * jnp.tanh(c)
                hd.append(h)
            hs.append(hd)
        for t in range(STEPS):
            outT = jnp.concatenate([hs[0][t], hs[1][STEPS - 1 - t]], axis=0)
            lg = (jnp.dot(qlin_ref[...], outT, preferred_element_type=f32)
                  + qlb_ref[:, 0:1])
            m = jnp.max(lg, axis=0, keepdims=True)
            e = jnp.exp(lg - m)
            out_ref[r * STEPS + t] = e / jnp.sum(e, axis=0, keepdims=True)


def _query_attn_call(q_ohT, q_embT, q_Wih, q_Whh, qb, qlinP, qlbP):
    return pl.pallas_call(
        _query_body,
        in_specs=[
            pl.BlockSpec((32, B), lambda: (0, 0)),
            pl.BlockSpec((EMB, 32), lambda: (0, 0)),
            pl.BlockSpec((RANK, 2, 4 * HID, EMB), lambda: (0, 0, 0, 0)),
            pl.BlockSpec((RANK, 2, 4 * HID, HID), lambda: (0, 0, 0, 0)),
            pl.BlockSpec((RANK, 2, 4 * HID, 8), lambda: (0, 0, 0, 0)),
            pl.BlockSpec((32, 2 * HID), lambda: (0, 0)),
            pl.BlockSpec((32, 8), lambda: (0, 0)),
        ],
        out_specs=pl.BlockSpec((RANK * STEPS, 32, B), lambda: (0, 0, 0)),
        out_shape=jax.ShapeDtypeStruct((RANK * STEPS, 32, B), jnp.float32),
    )(q_ohT, q_embT, q_Wih, q_Whh, qb, qlinP, qlbP)


_DUMMY_PK = -1073741824


@functools.partial(
    pl.kernel,
    out_type=(jax.ShapeDtypeStruct((NTILE * TPF,), jnp.int32),
              jax.ShapeDtypeStruct((NTILE * TPF,), jnp.float32),
              jax.ShapeDtypeStruct((NTILE * 16,), jnp.int32)),
    mesh=_SC_MESH,
    compiler_params=_SC_PARAMS,
    scratch_types=[
        pltpu.VMEM((TPF * 3,), jnp.int32),
        pltpu.VMEM((TPF + 16,), jnp.int32),
        pltpu.VMEM((TPF + 16,), jnp.int32),
        pltpu.VMEM((TPF,), jnp.float32),
        pltpu.VMEM((16,), jnp.int32),
        pltpu.SemaphoreType.DMA,
    ],
)
def _pack_kernel(facts_hbm, ea_hbm, pk_hbm, vl_hbm, cnt_hbm,
                 f_v, idx_v, pk_v, vl_v, cnt_v, sem):
    wid = lax.axis_index("s") * 2 + lax.axis_index("c")
    true_start = wid * TPF
    true_end = jnp.minimum(true_start + TPF, NF)
    buf_base = jnp.minimum(true_start, NF - TPF)
    g0 = (true_start - buf_base) // 16
    g1 = g0 + (true_end - true_start) // 16
    pltpu.sync_copy(facts_hbm.at[pl.ds(buf_base * 3, TPF * 3)], f_v)
    lanes = lax.iota(jnp.int32, 16)
    dummy16 = jnp.full((16,), _DUMMY_PK, jnp.int32)
    zi16 = jnp.zeros(16, jnp.int32)

    @plsc.parallel_loop(0, TPF // 16 + 1, 1, unroll=3)
    def prefill(g):
        pk_v[pl.ds(g * 16, 16)] = dummy16
        idx_v[pl.ds(g * 16, 16)] = zi16

    def grp(g, off_vec):
        row = (g * 16 + lanes) * 3
        rel = plsc.load_gather(f_v, [row])
        fh = plsc.load_gather(f_v, [row + 1])
        ft = plsc.load_gather(f_v, [row + 2])
        mask = rel < 12
        mi = mask.astype(jnp.int32)
        pos = off_vec + plsc.cumsum(mi) - mi
        pk = fh | (ft << 14) | (rel << 28)
        plsc.store_scatter(pk_v, [pos], pk, mask=mask)
        plsc.store_scatter(idx_v, [pos], fh * N_OPS + rel, mask=mask)
        return off_vec + plsc.all_reduce_population_count(mask)

    off_vec = lax.fori_loop(g0, g1, grp, jnp.zeros((16,), jnp.int32))
    ng = (off_vec[0] + 15) // 16
    cnt_v[...] = jnp.broadcast_to(ng, (16,))
    pltpu.async_copy(ea_hbm.at[idx_v.at[pl.ds(0, TPF)]], vl_v, sem).wait()
    pltpu.sync_copy(pk_v.at[pl.ds(0, TPF)], pk_hbm.at[pl.ds(wid * TPF, TPF)])
    pltpu.sync_copy(vl_v, vl_hbm.at[pl.ds(wid * TPF, TPF)])
    pltpu.sync_copy(cnt_v, cnt_hbm.at[pl.ds(wid * 16, 16)])


@functools.partial(
    pl.kernel,
    out_type=jax.ShapeDtypeStruct((B, N_ENT), jnp.float32),
    mesh=_SC_MESH,
    compiler_params=_SC_PARAMS,
    scratch_types=[
        pltpu.VMEM((N_ENT,), jnp.float32),
        pltpu.VMEM((N_ENT,), jnp.float32),
        pltpu.VMEM((N_ENT,), jnp.float32),
        pltpu.VMEM((N_ENT,), jnp.float32),
        pltpu.VMEM((N_ENT,), jnp.float32),
        pltpu.VMEM((N_ENT,), jnp.float32),
        pltpu.VMEM((4 * TPF,), jnp.int32),
        pltpu.VMEM((4 * TPF,), jnp.float32),
        pltpu.VMEM((64,), jnp.float32),
        pltpu.VMEM((B,), jnp.int32),
        pltpu.VMEM((NTILE * 16,), jnp.int32),
        pltpu.SemaphoreType.DMA,
        pltpu.SemaphoreType.DMA,
    ],
)
def _prop_kernel(pk_hbm, vl_hbm, cnt_hbm, tbl_hbm, heads_hbm, out_hbm,
                 M0, M1, A0, A1, L0, L1, pk_v, vl_v, tbl_v, heads_v, cnt_v,
                 sem_pk, sem_vl):
    f32 = jnp.float32
    wid = lax.axis_index("s") * 2 + lax.axis_index("c")
    lanes = lax.iota(jnp.int32, 16)
    lane0 = lanes == 0
    NV = N_ENT // 16
    z16 = jnp.zeros(16, f32)
    ones16 = jnp.ones(16, f32)

    pltpu.sync_copy(heads_hbm, heads_v)
    pltpu.sync_copy(cnt_hbm, cnt_v)
    h0v = plsc.load_gather(heads_v, [jnp.full((16,), 0, jnp.int32) + 2 * wid])
    h1v = plsc.load_gather(heads_v,
                           [jnp.full((16,), 1, jnp.int32) + 2 * wid])

    def fact_sweep(accs):
        for pb in range(3):
            pltpu.make_async_copy(pk_hbm.at[pl.ds(pb * TPF, TPF)],
                                  pk_v.at[pl.ds(pb * TPF, TPF)],
                                  sem_pk).start()
            pltpu.make_async_copy(vl_hbm.at[pl.ds(pb * TPF, TPF)],
                                  vl_v.at[pl.ds(pb * TPF, TPF)],
                                  sem_vl).start()

        def chunk(ci, a):
            par = lax.rem(ci, 4) * TPF
            ng = cnt_v[pl.ds(ci * 16, 16)][0]
            pltpu.make_async_copy(pk_hbm.at[pl.ds(0, TPF)],
                                  pk_v.at[pl.ds(par, TPF)], sem_pk).wait()
            pltpu.make_async_copy(vl_hbm.at[pl.ds(0, TPF)],
                                  vl_v.at[pl.ds(par, TPF)], sem_vl).wait()

            @pl.when(ci + 3 < NTILE)
            def _():
                nxt = lax.rem(ci + 3, 4) * TPF
                pltpu.make_async_copy(
                    pk_hbm.at[pl.ds((ci + 3) * TPF, TPF)],
                    pk_v.at[pl.ds(nxt, TPF)], sem_pk).start()
                pltpu.make_async_copy(
                    vl_hbm.at[pl.ds((ci + 3) * TPF, TPF)],
                    vl_v.at[pl.ds(nxt, TPF)], sem_vl).start()

            @plsc.parallel_loop(0, ng, 1, unroll=2, carry=a)
            def grp(g, acc):
                a0, a1 = acc
                pk = pk_v[pl.ds(par + g * 16, 16)]
                vl = vl_v[pl.ds(par + g * 16, 16)]
                fh = pk & 0x3FFF
                ft = (pk >> 14) & 0x3FFF
                rl = (pk >> 28) & 0xF
                cf0 = vl * plsc.load_gather(tbl_v, [rl])
                cb0 = vl * plsc.load_gather(tbl_v, [rl + 16])
                cf1 = vl * plsc.load_gather(tbl_v, [rl + 32])
                cb1 = vl * plsc.load_gather(tbl_v, [rl + 48])
                m0h = plsc.load_gather(M0, [fh])
                m0t = plsc.load_gather(M0, [ft])
                m1h = plsc.load_gather(M1, [fh])
                m1t = plsc.load_gather(M1, [ft])
                p0f = m0h * cf0
                p0b = m0t * cb0
                p1f = m1h * cf1
                p1b = m1t * cb1
                plsc.addupdate_scatter(A0, [ft], p0f)
                plsc.addupdate_scatter(A0, [fh], p0b)
                plsc.addupdate_scatter(A1, [ft], p1f)
                plsc.addupdate_scatter(A1, [fh], p1b)
                return (a0 + (p0f + p0b), a1 + (p1f + p1b))

            return grp

        return lax.fori_loop(0, NTILE, chunk, accs)

    def load_tbl(r, t):
        ti = r * STEPS + t
        pltpu.sync_copy(tbl_hbm.at[ti * NTILE + wid], tbl_v)
        a24_0 = plsc.load_gather(tbl_v, [jnp.full((16,), 15, jnp.int32)])
        a24_1 = plsc.load_gather(tbl_v, [jnp.full((16,), 47, jnp.int32)])
        return a24_0, a24_1

    for r in range(RANK):
        a24_0, a24_1 = load_tbl(r, 0)

        @plsc.parallel_loop(0, NV, 1, unroll=5)
        def zinit(i):
            M0[pl.ds(i * 16, 16)] = z16
            M1[pl.ds(i * 16, 16)] = z16
            A0[pl.ds(i * 16, 16)] = z16
            A1[pl.ds(i * 16, 16)] = z16

        plsc.store_scatter(M0, [h0v], ones16, mask=lane0)
        plsc.store_scatter(M1, [h1v], ones16, mask=lane0)
        plsc.store_scatter(A0, [h0v], a24_0, mask=lane0)
        plsc.store_scatter(A1, [h1v], a24_1, mask=lane0)
        sm0 = ones16
        sm1 = ones16

        for t in range(STEPS):
            acc0, acc1 = fact_sweep((z16, z16))

            nr0 = a24_0 * sm0 + jnp.broadcast_to(jnp.sum(acc0), (16,))
            nr1 = a24_1 * sm1 + jnp.broadcast_to(jnp.sum(acc1), (16,))
            rc0 = 1.0 / jnp.maximum(nr0, 1e-20)
            rc1 = 1.0 / jnp.maximum(nr1, 1e-20)
            sm0 = nr0 * rc0
            sm1 = nr1 * rc1

            if t < STEPS - 1:
                a24_0, a24_1 = load_tbl(r, t + 1)

                @plsc.parallel_loop(0, NV, 1, unroll=5)
                def renorm(i):
                    m0 = A0[pl.ds(i * 16, 16)] * rc0
                    m1 = A1[pl.ds(i * 16, 16)] * rc1
                    M0[pl.ds(i * 16, 16)] = m0
                    M1[pl.ds(i * 16, 16)] = m1
                    A0[pl.ds(i * 16, 16)] = a24_0 * m0
                    A1[pl.ds(i * 16, 16)] = a24_1 * m1
            elif r == 0:
                @plsc.parallel_loop(0, NV, 1, unroll=5)
                def linit(i):
                    L0[pl.ds(i * 16, 16)] = A0[pl.ds(i * 16, 16)] * rc0
                    L1[pl.ds(i * 16, 16)] = A1[pl.ds(i * 16, 16)] * rc1
            else:
                @plsc.parallel_loop(0, NV, 1, unroll=5)
                def lacc(i):
                    L0[pl.ds(i * 16, 16)] = (L0[pl.ds(i * 16, 16)]
                                             + A0[pl.ds(i * 16, 16)] * rc0)
                    L1[pl.ds(i * 16, 16)] = (L1[pl.ds(i * 16, 16)]
                                             + A1[pl.ds(i * 16, 16)] * rc1)

    pltpu.sync_copy(L0, out_hbm.at[2 * wid])
    pltpu.sync_copy(L1, out_hbm.at[2 * wid + 1])


def kernel(queries, heads, facts, entity_degrees, query_emb, entity_emb,
           q_Wih, q_Whh, q_bih, q_bhh, e_Wih, e_Whh, e_bih, e_bhh,
           qlin_W, qlin_b, elin_W, elin_b):
    f32 = jnp.float32

    emb_pad = jnp.zeros((32, EMB), f32).at[:N_OPS + 1].set(entity_emb)
    ebias = jnp.zeros((2, 8, 4 * HID), f32).at[:, 0, :].set(e_bih + e_bhh)
    elbP = jnp.zeros((8, N_OPS), f32).at[0].set(elin_b)
    ea = _entity_lstm_call(entity_degrees.astype(jnp.int32), emb_pad, e_Wih,
                           e_Whh, ebias, elin_W, elbP)

    q_ohT = jax.nn.one_hot(queries, 32, axis=0, dtype=f32)
    q_embT = jnp.zeros((EMB, 32), f32).at[:, :N_OPS].set(query_emb.T)
    qb = jnp.zeros((RANK, 2, 4 * HID, 8), f32).at[:, :, :, 0].set(
        q_bih + q_bhh)
    qlinP = jnp.zeros((32, 2 * HID), f32).at[:N_OPS + 1].set(qlin_W)
    qlbP = jnp.full((32, 8), -1e30, f32).at[:N_OPS + 1, :].set(
        qlin_b[:, None])
    attnT = _query_attn_call(q_ohT, q_embT, q_Wih, q_Whh, qb, qlinP, qlbP)

    fwd = attnT[:, 0:12, :]
    bwd = attnT[:, 12:24, :]
    a24 = attnT[:, 24:25, :]
    zero3 = jnp.zeros((RANK * STEPS, 3, B), f32)
    zero4 = jnp.zeros((RANK * STEPS, 4, B), f32)
    tbl = jnp.concatenate([fwd, zero3, a24, bwd, zero4], axis=1)
    tbl = jnp.transpose(tbl, (0, 2, 1)).reshape(RANK * STEPS * NTILE, 64)

    pk, vl, cnt = _pack_kernel(facts.astype(jnp.int32).reshape(-1),
                               ea.reshape(-1))

    return _prop_kernel(pk, vl, cnt, tbl, heads.astype(jnp.int32))
